# Initial kernel scaffold; baseline (speedup 1.0000x reference)
import jax
import jax.numpy as jnp
from jax.experimental import pallas as pl


def kernel(hidden_states, W_gate):
    raise NotImplementedError("write your pallas kernel here")


# speedup vs baseline: 4.5680x; 4.5680x over previous
import jax
import jax.numpy as jnp
from jax.experimental import pallas as pl

_TOKEN_BLOCK = 1024


def _router_kernel(x_ref, w_ref, gate_ref, idx_ref, logits_ref):
    x = x_ref[...]
    w = w_ref[...]
    logits = jax.lax.dot_general(
        x, w, (((1,), (1,)), ((), ())), preferred_element_type=jnp.float32)
    logits_ref[...] = logits

    m = jnp.max(logits, axis=1, keepdims=True)
    e = jnp.exp(logits - m)
    z = jnp.sum(e, axis=1, keepdims=True)
    p = e / z

    num_e = p.shape[1]
    iota = jax.lax.broadcasted_iota(jnp.int32, p.shape, 1)

    m1 = jnp.max(p, axis=1, keepdims=True)
    i1 = jnp.min(jnp.where(p == m1, iota, num_e), axis=1, keepdims=True)
    p_masked = jnp.where(iota == i1, -jnp.inf, p)
    m2 = jnp.max(p_masked, axis=1, keepdims=True)
    i2 = jnp.min(jnp.where(p_masked == m2, iota, num_e), axis=1, keepdims=True)

    denom = m1 + m2 + 1e-9
    w1 = m1 / denom
    w2 = m2 / denom

    gate_ref[...] = jnp.where(iota == i1, w1,
                              jnp.where(iota == i2, w2, jnp.float32(0.0)))
    idx_ref[...] = jnp.concatenate([i1, i2], axis=1)


def kernel(hidden_states, W_gate):
    B, S, H = hidden_states.shape
    E = W_gate.shape[0]
    N = B * S
    T = _TOKEN_BLOCK
    x = hidden_states.reshape(N, H)
    gate, idx, logits = pl.pallas_call(
        _router_kernel,
        grid=(N // T,),
        in_specs=[
            pl.BlockSpec((T, H), lambda i: (i, 0)),
            pl.BlockSpec((E, H), lambda i: (0, 0)),
        ],
        out_specs=[
            pl.BlockSpec((T, E), lambda i: (i, 0)),
            pl.BlockSpec((T, 2), lambda i: (i, 0)),
            pl.BlockSpec((T, E), lambda i: (i, 0)),
        ],
        out_shape=[
            jax.ShapeDtypeStruct((N, E), jnp.float32),
            jax.ShapeDtypeStruct((N, 2), jnp.int32),
            jax.ShapeDtypeStruct((N, E), jnp.float32),
        ],
    )(x, W_gate)
    return (gate.reshape(B, S, E), idx.reshape(B, S, 2),
            logits.reshape(B, S, E))


# speedup vs baseline: 4.8989x; 1.0725x over previous
import jax
import jax.numpy as jnp
from jax.experimental import pallas as pl

_TOKEN_BLOCK = 1024


def _router_kernel(x_ref, w_ref, gate_ref, idx_ref, logits_ref):
    x = x_ref[...]
    w = w_ref[...]
    logits = jax.lax.dot_general(
        x, w, (((1,), (1,)), ((), ())), preferred_element_type=jnp.float32)
    logits_ref[...] = logits

    num_e = logits.shape[1]
    iota_f = jax.lax.broadcasted_iota(
        jnp.int32, logits.shape, 1).astype(jnp.float32)

    m1 = jnp.max(logits, axis=1, keepdims=True)
    i1 = jnp.min(jnp.where(logits == m1, iota_f, float(num_e)),
                 axis=1, keepdims=True)
    sel1 = iota_f == i1
    l_masked = jnp.where(sel1, -jnp.inf, logits)
    m2 = jnp.max(l_masked, axis=1, keepdims=True)
    i2 = jnp.min(jnp.where(l_masked == m2, iota_f, float(num_e)),
                 axis=1, keepdims=True)

    z = jnp.sum(jnp.exp(logits - m1), axis=1, keepdims=True)
    p1 = 1.0 / z
    p2 = jnp.exp(m2 - m1) / z
    denom = p1 + p2 + 1e-9
    w1 = p1 / denom
    w2 = p2 / denom

    gate_ref[...] = jnp.where(sel1, w1,
                              jnp.where(iota_f == i2, w2, jnp.float32(0.0)))
    idx_ref[...] = jnp.concatenate(
        [i1.astype(jnp.int32), i2.astype(jnp.int32)], axis=1)


def kernel(hidden_states, W_gate):
    B, S, H = hidden_states.shape
    E = W_gate.shape[0]
    N = B * S
    T = _TOKEN_BLOCK
    x = hidden_states.reshape(N, H)
    gate, idx, logits = pl.pallas_call(
        _router_kernel,
        grid=(N // T,),
        in_specs=[
            pl.BlockSpec((T, H), lambda i: (i, 0)),
            pl.BlockSpec((E, H), lambda i: (0, 0)),
        ],
        out_specs=[
            pl.BlockSpec((T, E), lambda i: (i, 0)),
            pl.BlockSpec((T, 2), lambda i: (i, 0)),
            pl.BlockSpec((T, E), lambda i: (i, 0)),
        ],
        out_shape=[
            jax.ShapeDtypeStruct((N, E), jnp.float32),
            jax.ShapeDtypeStruct((N, 2), jnp.int32),
            jax.ShapeDtypeStruct((N, E), jnp.float32),
        ],
    )(x, W_gate)
    return (gate.reshape(B, S, E), idx.reshape(B, S, 2),
            logits.reshape(B, S, E))


# speedup vs baseline: 5.0658x; 1.0341x over previous
import jax
import jax.numpy as jnp
from jax.experimental import pallas as pl

_TOKEN_BLOCK = 1024


def _router_kernel(x_ref, w_ref, gate_ref, idx_ref, logits_ref):
    x = x_ref[0]
    w = w_ref[...]
    logits = jax.lax.dot_general(
        x, w, (((1,), (1,)), ((), ())), preferred_element_type=jnp.float32)
    logits_ref[0] = logits

    num_e = logits.shape[1]
    iota_f = jax.lax.broadcasted_iota(
        jnp.int32, logits.shape, 1).astype(jnp.float32)

    m1 = jnp.max(logits, axis=1, keepdims=True)
    i1 = jnp.min(jnp.where(logits == m1, iota_f, float(num_e)),
                 axis=1, keepdims=True)
    sel1 = iota_f == i1
    l_masked = jnp.where(sel1, -jnp.inf, logits)
    m2 = jnp.max(l_masked, axis=1, keepdims=True)
    i2 = jnp.min(jnp.where(l_masked == m2, iota_f, float(num_e)),
                 axis=1, keepdims=True)

    z = jnp.sum(jnp.exp(logits - m1), axis=1, keepdims=True)
    p1 = 1.0 / z
    p2 = jnp.exp(m2 - m1) / z
    denom = p1 + p2 + 1e-9
    w1 = p1 / denom
    w2 = p2 / denom

    gate_ref[0] = jnp.where(sel1, w1,
                            jnp.where(iota_f == i2, w2, jnp.float32(0.0)))
    idx_ref[0] = jnp.concatenate(
        [i1.astype(jnp.int32), i2.astype(jnp.int32)], axis=1)


def kernel(hidden_states, W_gate):
    B, S, H = hidden_states.shape
    E = W_gate.shape[0]
    T = _TOKEN_BLOCK
    gate, idx, logits = pl.pallas_call(
        _router_kernel,
        grid=(B, S // T),
        in_specs=[
            pl.BlockSpec((1, T, H), lambda b, s: (b, s, 0)),
            pl.BlockSpec((E, H), lambda b, s: (0, 0)),
        ],
        out_specs=[
            pl.BlockSpec((1, T, E), lambda b, s: (b, s, 0)),
            pl.BlockSpec((1, T, 2), lambda b, s: (b, s, 0)),
            pl.BlockSpec((1, T, E), lambda b, s: (b, s, 0)),
        ],
        out_shape=[
            jax.ShapeDtypeStruct((B, S, E), jnp.float32),
            jax.ShapeDtypeStruct((B, S, 2), jnp.int32),
            jax.ShapeDtypeStruct((B, S, E), jnp.float32),
        ],
    )(hidden_states, W_gate)
    return (gate, idx, logits)


# speedup vs baseline: 5.5092x; 1.0875x over previous
import jax
import jax.numpy as jnp
from jax.experimental import pallas as pl

_TOKEN_BLOCK = 2048


def _router_kernel(x_ref, w_ref, gate_ref, idx_ref, logits_ref):
    x = x_ref[0]
    w = w_ref[...]
    logits = jax.lax.dot_general(
        x, w, (((1,), (1,)), ((), ())), preferred_element_type=jnp.float32)
    logits_ref[0] = logits

    num_e = logits.shape[1]
    iota_f = jax.lax.broadcasted_iota(
        jnp.int32, logits.shape, 1).astype(jnp.float32)

    m1 = jnp.max(logits, axis=1, keepdims=True)
    i1 = jnp.min(jnp.where(logits == m1, iota_f, float(num_e)),
                 axis=1, keepdims=True)
    sel1 = iota_f == i1
    l_masked = jnp.where(sel1, -jnp.inf, logits)
    m2 = jnp.max(l_masked, axis=1, keepdims=True)
    i2 = jnp.min(jnp.where(l_masked == m2, iota_f, float(num_e)),
                 axis=1, keepdims=True)

    z = jnp.sum(jnp.exp(logits - m1), axis=1, keepdims=True)
    p1 = 1.0 / z
    p2 = jnp.exp(m2 - m1) / z
    denom = p1 + p2 + 1e-9
    w1 = p1 / denom
    w2 = p2 / denom

    gate_ref[0] = jnp.where(sel1, w1,
                            jnp.where(iota_f == i2, w2, jnp.float32(0.0)))
    idx_ref[0] = jnp.concatenate(
        [i1.astype(jnp.int32), i2.astype(jnp.int32)], axis=1)


def kernel(hidden_states, W_gate):
    B, S, H = hidden_states.shape
    E = W_gate.shape[0]
    T = _TOKEN_BLOCK
    gate, idx, logits = pl.pallas_call(
        _router_kernel,
        grid=(B, S // T),
        in_specs=[
            pl.BlockSpec((1, T, H), lambda b, s: (b, s, 0)),
            pl.BlockSpec((E, H), lambda b, s: (0, 0)),
        ],
        out_specs=[
            pl.BlockSpec((1, T, E), lambda b, s: (b, s, 0)),
            pl.BlockSpec((1, T, 2), lambda b, s: (b, s, 0)),
            pl.BlockSpec((1, T, E), lambda b, s: (b, s, 0)),
        ],
        out_shape=[
            jax.ShapeDtypeStruct((B, S, E), jnp.float32),
            jax.ShapeDtypeStruct((B, S, 2), jnp.int32),
            jax.ShapeDtypeStruct((B, S, E), jnp.float32),
        ],
    )(hidden_states, W_gate)
    return (gate, idx, logits)


# speedup vs baseline: 5.8395x; 1.0599x over previous
import jax
import jax.numpy as jnp
from jax.experimental import pallas as pl

_TOKEN_BLOCK = 4096


def _router_kernel(x_ref, w_ref, gate_ref, idx_ref, logits_ref):
    x = x_ref[0]
    w = w_ref[...]
    logits = jax.lax.dot_general(
        x, w, (((1,), (1,)), ((), ())), preferred_element_type=jnp.float32)
    logits_ref[0] = logits

    num_e = logits.shape[1]
    iota_f = jax.lax.broadcasted_iota(
        jnp.int32, logits.shape, 1).astype(jnp.float32)

    m1 = jnp.max(logits, axis=1, keepdims=True)
    i1 = jnp.min(jnp.where(logits == m1, iota_f, float(num_e)),
                 axis=1, keepdims=True)
    sel1 = iota_f == i1
    l_masked = jnp.where(sel1, -jnp.inf, logits)
    m2 = jnp.max(l_masked, axis=1, keepdims=True)
    i2 = jnp.min(jnp.where(l_masked == m2, iota_f, float(num_e)),
                 axis=1, keepdims=True)

    z = jnp.sum(jnp.exp(logits - m1), axis=1, keepdims=True)
    p1 = 1.0 / z
    p2 = jnp.exp(m2 - m1) / z
    denom = p1 + p2 + 1e-9
    w1 = p1 / denom
    w2 = p2 / denom

    gate_ref[0] = jnp.where(sel1, w1,
                            jnp.where(iota_f == i2, w2, jnp.float32(0.0)))
    idx_ref[0] = jnp.concatenate(
        [i1.astype(jnp.int32), i2.astype(jnp.int32)], axis=1)


def kernel(hidden_states, W_gate):
    B, S, H = hidden_states.shape
    E = W_gate.shape[0]
    T = _TOKEN_BLOCK
    gate, idx, logits = pl.pallas_call(
        _router_kernel,
        grid=(B, S // T),
        in_specs=[
            pl.BlockSpec((1, T, H), lambda b, s: (b, s, 0)),
            pl.BlockSpec((E, H), lambda b, s: (0, 0)),
        ],
        out_specs=[
            pl.BlockSpec((1, T, E), lambda b, s: (b, s, 0)),
            pl.BlockSpec((1, T, 2), lambda b, s: (b, s, 0)),
            pl.BlockSpec((1, T, E), lambda b, s: (b, s, 0)),
        ],
        out_shape=[
            jax.ShapeDtypeStruct((B, S, E), jnp.float32),
            jax.ShapeDtypeStruct((B, S, 2), jnp.int32),
            jax.ShapeDtypeStruct((B, S, E), jnp.float32),
        ],
    )(hidden_states, W_gate)
    return (gate, idx, logits)


# speedup vs baseline: 11.9046x; 2.0386x over previous
import jax
import jax.numpy as jnp
from jax.experimental import pallas as pl

_TOKEN_BLOCK = 4096


def _router_kernel(x_ref, w_ref, gate_ref, code_ref, logits_ref):
    x = x_ref[0]
    w = w_ref[...]
    logits = jax.lax.dot_general(
        x, w, (((1,), (1,)), ((), ())), preferred_element_type=jnp.float32)
    lt = logits.T
    logits_ref[0] = lt

    num_e = lt.shape[0]
    iota_f = jax.lax.broadcasted_iota(
        jnp.int32, lt.shape, 0).astype(jnp.float32)

    m1 = jnp.max(lt, axis=0, keepdims=True)
    i1 = jnp.min(jnp.where(lt == m1, iota_f, float(num_e)),
                 axis=0, keepdims=True)
    sel1 = iota_f == i1
    l_masked = jnp.where(sel1, -jnp.inf, lt)
    m2 = jnp.max(l_masked, axis=0, keepdims=True)
    i2 = jnp.min(jnp.where(l_masked == m2, iota_f, float(num_e)),
                 axis=0, keepdims=True)

    z = jnp.sum(jnp.exp(lt - m1), axis=0, keepdims=True)
    p1 = 1.0 / z
    p2 = jnp.exp(m2 - m1) / z
    denom = p1 + p2 + 1e-9
    w1 = p1 / denom
    w2 = p2 / denom

    gate_ref[0] = jnp.where(sel1, w1,
                            jnp.where(iota_f == i2, w2, jnp.float32(0.0)))
    code = (i1 * float(num_e) + i2).astype(jnp.int32)
    code_ref[0] = jnp.broadcast_to(code, (8, code.shape[1]))


def kernel(hidden_states, W_gate):
    B, S, H = hidden_states.shape
    E = W_gate.shape[0]
    T = _TOKEN_BLOCK
    gate_t, code8, logits_t = pl.pallas_call(
        _router_kernel,
        grid=(B, S // T),
        in_specs=[
            pl.BlockSpec((1, T, H), lambda b, s: (b, s, 0)),
            pl.BlockSpec((E, H), lambda b, s: (0, 0)),
        ],
        out_specs=[
            pl.BlockSpec((1, E, T), lambda b, s: (b, 0, s)),
            pl.BlockSpec((1, 8, T), lambda b, s: (b, 0, s)),
            pl.BlockSpec((1, E, T), lambda b, s: (b, 0, s)),
        ],
        out_shape=[
            jax.ShapeDtypeStruct((B, E, S), jnp.float32),
            jax.ShapeDtypeStruct((B, 8, S), jnp.int32),
            jax.ShapeDtypeStruct((B, E, S), jnp.float32),
        ],
    )(hidden_states, W_gate)
    gate = jnp.transpose(gate_t, (0, 2, 1))
    logits = jnp.transpose(logits_t, (0, 2, 1))
    code = code8[:, 0, :]
    idx = jnp.stack([code // E, code % E], axis=-1)
    return (gate, idx, logits)


# speedup vs baseline: 12.2646x; 1.0302x over previous
import jax
import jax.numpy as jnp
from jax.experimental import pallas as pl

_TOKEN_BLOCK = 4096


def _router_kernel(x_ref, w_ref, gate_ref, i1_ref, i2_ref, logits_ref):
    x = x_ref[0]
    w = w_ref[...]
    logits = jax.lax.dot_general(
        x, w, (((1,), (1,)), ((), ())), preferred_element_type=jnp.float32)
    lt = logits.T
    logits_ref[0] = lt

    num_e = lt.shape[0]
    iota_f = jax.lax.broadcasted_iota(
        jnp.int32, lt.shape, 0).astype(jnp.float32)

    m1 = jnp.max(lt, axis=0, keepdims=True)
    i1 = jnp.min(jnp.where(lt == m1, iota_f, float(num_e)),
                 axis=0, keepdims=True)
    sel1 = iota_f == i1
    l_masked = jnp.where(sel1, -jnp.inf, lt)
    m2 = jnp.max(l_masked, axis=0, keepdims=True)
    i2 = jnp.min(jnp.where(l_masked == m2, iota_f, float(num_e)),
                 axis=0, keepdims=True)

    z = jnp.sum(jnp.exp(lt - m1), axis=0, keepdims=True)
    p1 = 1.0 / z
    p2 = jnp.exp(m2 - m1) / z
    denom = p1 + p2 + 1e-9
    w1 = p1 / denom
    w2 = p2 / denom

    gate_ref[0] = jnp.where(sel1, w1,
                            jnp.where(iota_f == i2, w2, jnp.float32(0.0)))
    i1_ref[0] = i1.astype(jnp.int32)
    i2_ref[0] = i2.astype(jnp.int32)


def kernel(hidden_states, W_gate):
    B, S, H = hidden_states.shape
    E = W_gate.shape[0]
    T = _TOKEN_BLOCK
    gate_t, i1r, i2r, logits_t = pl.pallas_call(
        _router_kernel,
        grid=(B, S // T),
        in_specs=[
            pl.BlockSpec((1, T, H), lambda b, s: (b, s, 0)),
            pl.BlockSpec((E, H), lambda b, s: (0, 0)),
        ],
        out_specs=[
            pl.BlockSpec((1, E, T), lambda b, s: (b, 0, s)),
            pl.BlockSpec((1, 1, T), lambda b, s: (b, 0, s)),
            pl.BlockSpec((1, 1, T), lambda b, s: (b, 0, s)),
            pl.BlockSpec((1, E, T), lambda b, s: (b, 0, s)),
        ],
        out_shape=[
            jax.ShapeDtypeStruct((B, E, S), jnp.float32),
            jax.ShapeDtypeStruct((B, 1, S), jnp.int32),
            jax.ShapeDtypeStruct((B, 1, S), jnp.int32),
            jax.ShapeDtypeStruct((B, E, S), jnp.float32),
        ],
    )(hidden_states, W_gate)
    gate = jnp.transpose(gate_t, (0, 2, 1))
    logits = jnp.transpose(logits_t, (0, 2, 1))
    idx = jnp.stack([i1r[:, 0, :], i2r[:, 0, :]], axis=-1)
    return (gate, idx, logits)
